# baseline (speedup 1.0000x reference)
import jax
import jax.numpy as jnp
from jax.experimental import pallas as pl
from jax.experimental.pallas import tpu as pltpu

_EPS = 1e-5
_VMEM_LIMIT = 100 * 1024 * 1024
_LP = 2


def _round_up(x, m):
    return (x + m - 1) // m * m


def _make_conv1_kernel(h, w, cin, cout, wp, th):
    n_h = h // th

    def kernel(x_ref, w_ref, out_ref, stats_ref, xs_ref):
        out_ref[0, 0:1, :, :] = jnp.zeros((1, wp, cout), jnp.bfloat16)
        out_ref[0, h + 1:h + 2, :, :] = jnp.zeros((1, wp, cout), jnp.bfloat16)
        out_ref[0, :, 0:_LP, :] = jnp.zeros((h + 2, _LP, cout), jnp.bfloat16)
        out_ref[0, :, w + _LP:wp, :] = jnp.zeros((h + 2, wp - w - _LP, cout),
                                                 jnp.bfloat16)

        xt = x_ref[0].astype(jnp.bfloat16)
        xt = jnp.transpose(xt, (1, 0, 2))
        xt = jnp.transpose(xt, (0, 2, 1))
        xs_ref[0:1, :, :] = jnp.zeros((1, wp, cin), jnp.bfloat16)
        xs_ref[h + 1:h + 2, :, :] = jnp.zeros((1, wp, cin), jnp.bfloat16)
        xs_ref[:, 0:_LP, :] = jnp.zeros((h + 2, _LP, cin), jnp.bfloat16)
        xs_ref[:, w + _LP:wp, :] = jnp.zeros((h + 2, wp - w - _LP, cin),
                                             jnp.bfloat16)
        xs_ref[1:h + 1, _LP:w + _LP, :] = xt

        def body(i, carry):
            s_acc, q_acc = carry
            slab = xs_ref[pl.ds(i * th, th + 2), :, :]
            taps = [slab[dy:dy + th, _LP - 1 + dx:_LP - 1 + dx + w, :]
                    for dy in range(3) for dx in range(3)]
            patches = jnp.concatenate(taps, axis=-1)
            patches = patches.reshape(th * w, 9 * cin)
            acc = jnp.dot(patches, w_ref[...],
                          preferred_element_type=jnp.float32)
            s_acc = s_acc + jnp.sum(acc, axis=0, keepdims=True)
            q_acc = q_acc + jnp.sum(acc * acc, axis=0, keepdims=True)
            out_ref[0, pl.ds(1 + i * th, th), _LP:w + _LP, :] = (
                acc.reshape(th, w, cout).astype(jnp.bfloat16))
            return s_acc, q_acc

        zero = jnp.zeros((1, cout), jnp.float32)
        s_acc, q_acc = jax.lax.fori_loop(0, n_h, body, (zero, zero),
                                         unroll=2)
        stats_ref[0, 0:1, :] = s_acc
        stats_ref[0, 1:2, :] = q_acc

    return kernel


def _make_conv2_kernel(h, w, cmid, cout, wp, th):
    n_h = h // th

    def kernel(c1_ref, w_ref, st1_ref, g1_ref, be1_ref, out_ref, stats_ref):
        inv_count = 1.0 / (st1_ref.shape[0] * h * w)
        st = jnp.sum(st1_ref[...], axis=0)
        mean = st[0:1, :] * inv_count
        var = jnp.maximum(st[1:2, :] * inv_count - mean * mean, 0.0)
        scale_v = g1_ref[...] * jax.lax.rsqrt(var + _EPS)
        shift_v = be1_ref[...] - mean * scale_v
        col_id = jax.lax.broadcasted_iota(jnp.int32, (th + 2, wp, cmid), 1)
        colmask = jnp.where((col_id >= _LP) & (col_id < w + _LP), 1.0, 0.0)
        scale = scale_v.reshape(1, 1, cmid) * colmask
        shift = shift_v.reshape(1, 1, cmid) * colmask
        zrow = jnp.zeros((1, wp, cmid), jnp.bfloat16)

        def tile(i, carry, top, bottom):
            s_acc, q_acc = carry
            slab = c1_ref[0, pl.ds(i * th, th + 2), :, :].astype(jnp.float32)
            hval = jnp.maximum(slab * scale + shift, 0.0).astype(jnp.bfloat16)
            if top:
                hval = jnp.concatenate([zrow, hval[1:]], axis=0)
            if bottom:
                hval = jnp.concatenate([hval[:th + 1], zrow], axis=0)
            taps = [hval[dy:dy + th, _LP - 1 + dx:_LP - 1 + dx + w, :]
                    for dy in range(3) for dx in range(3)]
            patches = jnp.concatenate(taps, axis=-1).reshape(th * w, 9 * cmid)
            res = jax.lax.dot_general(
                w_ref[...], patches,
                dimension_numbers=(((0,), (1,)), ((), ())),
                preferred_element_type=jnp.float32)
            s_acc = s_acc + jnp.sum(res, axis=1, keepdims=True)
            q_acc = q_acc + jnp.sum(res * res, axis=1, keepdims=True)
            out_ref[0, :, pl.ds(i * th * w, th * w)] = res.astype(jnp.bfloat16)
            return s_acc, q_acc

        zero = jnp.zeros((cout, 1), jnp.float32)
        carry = tile(0, (zero, zero), top=True, bottom=False)
        carry = jax.lax.fori_loop(
            1, n_h - 1,
            lambda i, c: tile(i, c, top=False, bottom=False),
            carry, unroll=2)
        s_acc, q_acc = tile(n_h - 1, carry, top=False, bottom=True)
        stats_ref[0, :, 0:1] = s_acc
        stats_ref[0, :, 1:2] = q_acc

    return kernel


def _make_bn2_relu_kernel(count):
    def kernel(c2_ref, st2_ref, g2_ref, be2_ref, out_ref):
        inv_count = 1.0 / count
        st = jnp.sum(st2_ref[...], axis=0)
        mean = st[:, 0:1] * inv_count
        var = jnp.maximum(st[:, 1:2] * inv_count - mean * mean, 0.0)
        scale = g2_ref[...] * jax.lax.rsqrt(var + _EPS)
        shift = be2_ref[...] - mean * scale
        out_ref[0] = jnp.maximum(
            c2_ref[0].astype(jnp.float32) * scale + shift, 0.0)
    return kernel


def kernel(x_nchw, w1, b1, g1, be1, w2, b2, g2, be2):
    n, cin, h, w = x_nchw.shape
    cout = w1.shape[-1]
    wp = _round_up(w + 2 * _LP, 16)
    th = 16 if (h % 16 == 0 and h >= 48) else 8

    w1f = w1.reshape(9 * cin, cout).astype(jnp.bfloat16)
    w2f = w2.reshape(9 * cout, cout).astype(jnp.bfloat16)

    c1, st1 = pl.pallas_call(
        _make_conv1_kernel(h, w, cin, cout, wp, th),
        grid=(n,),
        out_shape=(jax.ShapeDtypeStruct((n, h + 2, wp, cout), jnp.bfloat16),
                   jax.ShapeDtypeStruct((n, 2, cout), jnp.float32)),
        in_specs=[
            pl.BlockSpec((1, cin, h, w), lambda b: (b, 0, 0, 0)),
            pl.BlockSpec((9 * cin, cout), lambda b: (0, 0)),
        ],
        out_specs=(
            pl.BlockSpec((1, h + 2, wp, cout), lambda b: (b, 0, 0, 0)),
            pl.BlockSpec((1, 2, cout), lambda b: (b, 0, 0)),
        ),
        scratch_shapes=[pltpu.VMEM((h + 2, wp, cin), jnp.bfloat16)],
        compiler_params=pltpu.CompilerParams(
            dimension_semantics=("parallel",),
            vmem_limit_bytes=_VMEM_LIMIT),
    )(x_nchw, w1f)

    c2, st2 = pl.pallas_call(
        _make_conv2_kernel(h, w, cout, cout, wp, th),
        grid=(n,),
        out_shape=(jax.ShapeDtypeStruct((n, cout, h * w), jnp.bfloat16),
                   jax.ShapeDtypeStruct((n, cout, 2), jnp.float32)),
        in_specs=[
            pl.BlockSpec((1, h + 2, wp, cout), lambda b: (b, 0, 0, 0)),
            pl.BlockSpec((9 * cout, cout), lambda b: (0, 0)),
            pl.BlockSpec((n, 2, cout), lambda b: (0, 0, 0)),
            pl.BlockSpec((1, cout), lambda b: (0, 0)),
            pl.BlockSpec((1, cout), lambda b: (0, 0)),
        ],
        out_specs=(
            pl.BlockSpec((1, cout, h * w), lambda b: (b, 0, 0)),
            pl.BlockSpec((1, cout, 2), lambda b: (b, 0, 0)),
        ),
        compiler_params=pltpu.CompilerParams(
            dimension_semantics=("parallel",),
            vmem_limit_bytes=_VMEM_LIMIT),
    )(c1, w2f, st1, g1.reshape(1, cout).astype(jnp.float32),
      be1.reshape(1, cout).astype(jnp.float32))

    n_s = 2 if h % 2 == 0 else 1
    s_blk = h * w // n_s
    y = pl.pallas_call(
        _make_bn2_relu_kernel(float(n * h * w)),
        grid=(n, n_s),
        out_shape=jax.ShapeDtypeStruct((n, cout, h * w), jnp.float32),
        in_specs=[
            pl.BlockSpec((1, cout, s_blk), lambda b, j: (b, 0, j)),
            pl.BlockSpec((n, cout, 2), lambda b, j: (0, 0, 0)),
            pl.BlockSpec((cout, 1), lambda b, j: (0, 0)),
            pl.BlockSpec((cout, 1), lambda b, j: (0, 0)),
        ],
        out_specs=pl.BlockSpec((1, cout, s_blk), lambda b, j: (b, 0, j)),
        compiler_params=pltpu.CompilerParams(
            dimension_semantics=("parallel", "parallel"),
            vmem_limit_bytes=_VMEM_LIMIT),
    )(c2, st2, g2.reshape(cout, 1).astype(jnp.float32),
      be2.reshape(cout, 1).astype(jnp.float32))
    return y.reshape(n, cout, h, w)


# speedup vs baseline: 1.0983x; 1.0983x over previous
import jax
import jax.numpy as jnp
from jax.experimental import pallas as pl
from jax.experimental.pallas import tpu as pltpu

_EPS = 1e-5
_VMEM_LIMIT = 100 * 1024 * 1024
_LP = 2


def _round_up(x, m):
    return (x + m - 1) // m * m


def _make_conv1_kernel(h, w, cin, cout, wp, th):
    n_h = h // th

    def kernel(x_ref, w_ref, out_ref, stats_ref, xs_ref):
        out_ref[0, 0:1, :, :] = jnp.zeros((1, wp, cout), jnp.bfloat16)
        out_ref[0, h + 1:h + 2, :, :] = jnp.zeros((1, wp, cout), jnp.bfloat16)
        out_ref[0, :, 0:_LP, :] = jnp.zeros((h + 2, _LP, cout), jnp.bfloat16)
        out_ref[0, :, w + _LP:wp, :] = jnp.zeros((h + 2, wp - w - _LP, cout),
                                                 jnp.bfloat16)

        xt = x_ref[0].astype(jnp.bfloat16)
        xt = jnp.transpose(xt, (1, 0, 2))
        xt = jnp.transpose(xt, (0, 2, 1))
        xs_ref[0:1, :, :] = jnp.zeros((1, wp, cin), jnp.bfloat16)
        xs_ref[h + 1:h + 2, :, :] = jnp.zeros((1, wp, cin), jnp.bfloat16)
        xs_ref[:, 0:_LP, :] = jnp.zeros((h + 2, _LP, cin), jnp.bfloat16)
        xs_ref[:, w + _LP:wp, :] = jnp.zeros((h + 2, wp - w - _LP, cin),
                                             jnp.bfloat16)
        xs_ref[1:h + 1, _LP:w + _LP, :] = xt

        def body(i, carry):
            s_acc, q_acc = carry
            slab = xs_ref[pl.ds(i * th, th + 2), :, :]
            taps = [slab[dy:dy + th, _LP - 1 + dx:_LP - 1 + dx + w, :]
                    for dy in range(3) for dx in range(3)]
            patches = jnp.concatenate(taps, axis=-1)
            patches = patches.reshape(th * w, 9 * cin)
            acc = jnp.dot(patches, w_ref[...],
                          preferred_element_type=jnp.float32)
            s_acc = s_acc + jnp.sum(acc, axis=0, keepdims=True)
            q_acc = q_acc + jnp.sum(acc * acc, axis=0, keepdims=True)
            out_ref[0, pl.ds(1 + i * th, th), _LP:w + _LP, :] = (
                acc.reshape(th, w, cout).astype(jnp.bfloat16))
            return s_acc, q_acc

        zero = jnp.zeros((1, cout), jnp.float32)
        s_acc, q_acc = jax.lax.fori_loop(0, n_h, body, (zero, zero),
                                         unroll=2)
        stats_ref[0, 0:1, :] = s_acc
        stats_ref[0, 1:2, :] = q_acc

    return kernel


def _make_conv2_kernel(h, w, cmid, cout, wp, th):
    n_h = h // th

    def kernel(c1_ref, w_ref, st1_ref, g1_ref, be1_ref, out_ref, stats_ref):
        inv_count = 1.0 / (st1_ref.shape[0] * h * w)
        st = jnp.sum(st1_ref[...], axis=0)
        mean = st[0:1, :] * inv_count
        var = jnp.maximum(st[1:2, :] * inv_count - mean * mean, 0.0)
        scale_v = g1_ref[...] * jax.lax.rsqrt(var + _EPS)
        shift_v = be1_ref[...] - mean * scale_v
        col_id = jax.lax.broadcasted_iota(jnp.int32, (th + 2, wp, cmid), 1)
        colmask = jnp.where((col_id >= _LP) & (col_id < w + _LP), 1.0, 0.0)
        scale = scale_v.reshape(1, 1, cmid) * colmask
        shift = shift_v.reshape(1, 1, cmid) * colmask
        zrow = jnp.zeros((1, wp, cmid), jnp.bfloat16)

        def tile(i, carry, top, bottom):
            s_acc, q_acc = carry
            slab = c1_ref[0, pl.ds(i * th, th + 2), :, :].astype(jnp.float32)
            hval = jnp.maximum(slab * scale + shift, 0.0).astype(jnp.bfloat16)
            if top:
                hval = jnp.concatenate([zrow, hval[1:]], axis=0)
            if bottom:
                hval = jnp.concatenate([hval[:th + 1], zrow], axis=0)
            taps = [hval[dy:dy + th, _LP - 1 + dx:_LP - 1 + dx + w, :]
                    for dy in range(3) for dx in range(3)]
            patches = jnp.concatenate(taps, axis=-1).reshape(th * w, 9 * cmid)
            res = jax.lax.dot_general(
                w_ref[...], patches,
                dimension_numbers=(((0,), (1,)), ((), ())),
                preferred_element_type=jnp.float32)
            s_acc = s_acc + jnp.sum(res, axis=1, keepdims=True)
            q_acc = q_acc + jnp.sum(res * res, axis=1, keepdims=True)
            out_ref[0, :, pl.ds(i * th * w, th * w)] = res.astype(jnp.bfloat16)
            return s_acc, q_acc

        zero = jnp.zeros((cout, 1), jnp.float32)
        carry = tile(0, (zero, zero), top=True, bottom=False)
        carry = jax.lax.fori_loop(
            1, n_h - 1,
            lambda i, c: tile(i, c, top=False, bottom=False),
            carry, unroll=2)
        s_acc, q_acc = tile(n_h - 1, carry, top=False, bottom=True)
        stats_ref[0, :, 0:1] = s_acc
        stats_ref[0, :, 1:2] = q_acc

    return kernel


def _make_bn2_relu_kernel(count):
    def kernel(c2_ref, st2_ref, g2_ref, be2_ref, out_ref):
        c = g2_ref.shape[0]
        inv_count = 1.0 / count
        st = jnp.sum(st2_ref[...], axis=0)
        mean = st[:, 0:1] * inv_count
        var = jnp.maximum(st[:, 1:2] * inv_count - mean * mean, 0.0)
        scale = g2_ref[...] * jax.lax.rsqrt(var + _EPS)
        shift = be2_ref[...] - mean * scale
        out_ref[0] = jnp.maximum(
            c2_ref[0].astype(jnp.float32) * scale.reshape(c, 1, 1)
            + shift.reshape(c, 1, 1), 0.0)
    return kernel


def kernel(x_nchw, w1, b1, g1, be1, w2, b2, g2, be2):
    n, cin, h, w = x_nchw.shape
    cout = w1.shape[-1]
    wp = _round_up(w + 2 * _LP, 16)
    th = 16 if (h % 16 == 0 and h >= 48) else 8

    w1f = w1.reshape(9 * cin, cout).astype(jnp.bfloat16)
    w2f = w2.reshape(9 * cout, cout).astype(jnp.bfloat16)

    c1, st1 = pl.pallas_call(
        _make_conv1_kernel(h, w, cin, cout, wp, th),
        grid=(n,),
        out_shape=(jax.ShapeDtypeStruct((n, h + 2, wp, cout), jnp.bfloat16),
                   jax.ShapeDtypeStruct((n, 2, cout), jnp.float32)),
        in_specs=[
            pl.BlockSpec((1, cin, h, w), lambda b: (b, 0, 0, 0)),
            pl.BlockSpec((9 * cin, cout), lambda b: (0, 0)),
        ],
        out_specs=(
            pl.BlockSpec((1, h + 2, wp, cout), lambda b: (b, 0, 0, 0)),
            pl.BlockSpec((1, 2, cout), lambda b: (b, 0, 0)),
        ),
        scratch_shapes=[pltpu.VMEM((h + 2, wp, cin), jnp.bfloat16)],
        compiler_params=pltpu.CompilerParams(
            dimension_semantics=("parallel",),
            vmem_limit_bytes=_VMEM_LIMIT),
    )(x_nchw, w1f)

    c2, st2 = pl.pallas_call(
        _make_conv2_kernel(h, w, cout, cout, wp, th),
        grid=(n,),
        out_shape=(jax.ShapeDtypeStruct((n, cout, h * w), jnp.bfloat16),
                   jax.ShapeDtypeStruct((n, cout, 2), jnp.float32)),
        in_specs=[
            pl.BlockSpec((1, h + 2, wp, cout), lambda b: (b, 0, 0, 0)),
            pl.BlockSpec((9 * cout, cout), lambda b: (0, 0)),
            pl.BlockSpec((n, 2, cout), lambda b: (0, 0, 0)),
            pl.BlockSpec((1, cout), lambda b: (0, 0)),
            pl.BlockSpec((1, cout), lambda b: (0, 0)),
        ],
        out_specs=(
            pl.BlockSpec((1, cout, h * w), lambda b: (b, 0, 0)),
            pl.BlockSpec((1, cout, 2), lambda b: (b, 0, 0)),
        ),
        compiler_params=pltpu.CompilerParams(
            dimension_semantics=("parallel",),
            vmem_limit_bytes=_VMEM_LIMIT),
    )(c1, w2f, st1, g1.reshape(1, cout).astype(jnp.float32),
      be1.reshape(1, cout).astype(jnp.float32))

    c2_4d = c2.reshape(n, cout, h, w)

    n_s = 2 if h % 2 == 0 else 1
    h_blk = h // n_s
    return pl.pallas_call(
        _make_bn2_relu_kernel(float(n * h * w)),
        grid=(n, n_s),
        out_shape=jax.ShapeDtypeStruct((n, cout, h, w), jnp.float32),
        in_specs=[
            pl.BlockSpec((1, cout, h_blk, w), lambda b, j: (b, 0, j, 0)),
            pl.BlockSpec((n, cout, 2), lambda b, j: (0, 0, 0)),
            pl.BlockSpec((cout, 1), lambda b, j: (0, 0)),
            pl.BlockSpec((cout, 1), lambda b, j: (0, 0)),
        ],
        out_specs=pl.BlockSpec((1, cout, h_blk, w), lambda b, j: (b, 0, j, 0)),
        compiler_params=pltpu.CompilerParams(
            dimension_semantics=("parallel", "parallel"),
            vmem_limit_bytes=_VMEM_LIMIT),
    )(c2_4d, st2, g2.reshape(cout, 1).astype(jnp.float32),
      be2.reshape(cout, 1).astype(jnp.float32))


# speedup vs baseline: 1.2174x; 1.1085x over previous
import jax
import jax.numpy as jnp
from jax.experimental import pallas as pl
from jax.experimental.pallas import tpu as pltpu

_EPS = 1e-5
_VMEM_LIMIT = 100 * 1024 * 1024
_LP = 2


def _round_up(x, m):
    return (x + m - 1) // m * m


def _make_conv1_kernel(h, w, cin, cout, wp, th):
    n_h = h // th

    def kernel(x_ref, w_ref, out_ref, stats_ref, xs_ref):
        out_ref[0, 0:1, :, :] = jnp.zeros((1, wp, cout), jnp.bfloat16)
        out_ref[0, h + 1:h + 2, :, :] = jnp.zeros((1, wp, cout), jnp.bfloat16)
        out_ref[0, :, 0:_LP, :] = jnp.zeros((h + 2, _LP, cout), jnp.bfloat16)
        out_ref[0, :, w + _LP:wp, :] = jnp.zeros((h + 2, wp - w - _LP, cout),
                                                 jnp.bfloat16)

        xt = x_ref[0].astype(jnp.bfloat16)
        xt = jnp.transpose(xt, (1, 0, 2))
        xt = jnp.transpose(xt, (0, 2, 1))
        xs_ref[0:1, :, :] = jnp.zeros((1, wp, cin), jnp.bfloat16)
        xs_ref[h + 1:h + 2, :, :] = jnp.zeros((1, wp, cin), jnp.bfloat16)
        xs_ref[:, 0:_LP, :] = jnp.zeros((h + 2, _LP, cin), jnp.bfloat16)
        xs_ref[:, w + _LP:wp, :] = jnp.zeros((h + 2, wp - w - _LP, cin),
                                             jnp.bfloat16)
        xs_ref[1:h + 1, _LP:w + _LP, :] = xt

        def body(i, carry):
            s_acc, q_acc = carry
            slab = xs_ref[pl.ds(i * th, th + 2), :, :]
            taps = [slab[dy:dy + th, _LP - 1 + dx:_LP - 1 + dx + w, :]
                    for dy in range(3) for dx in range(3)]
            patches = jnp.concatenate(taps, axis=-1)
            patches = patches.reshape(th * w, 9 * cin)
            acc = jnp.dot(patches, w_ref[...],
                          preferred_element_type=jnp.float32)
            s_acc = s_acc + jnp.sum(acc, axis=0, keepdims=True)
            q_acc = q_acc + jnp.sum(acc * acc, axis=0, keepdims=True)
            out_ref[0, pl.ds(1 + i * th, th), _LP:w + _LP, :] = (
                acc.reshape(th, w, cout).astype(jnp.bfloat16))
            return s_acc, q_acc

        zero = jnp.zeros((1, cout), jnp.float32)
        s_acc, q_acc = jax.lax.fori_loop(0, n_h, body, (zero, zero),
                                         unroll=2)
        stats_ref[0, 0:1, :] = s_acc
        stats_ref[0, 1:2, :] = q_acc

    return kernel


def _make_conv2_kernel(h, w, cmid, cout, wp, th):
    n_h = h // th

    def kernel(c1_ref, w_ref, st1_ref, g1_ref, be1_ref, out_ref, stats_ref):
        inv_count = 1.0 / (st1_ref.shape[0] * h * w)
        st = jnp.sum(st1_ref[...], axis=0)
        mean = st[0:1, :] * inv_count
        var = jnp.maximum(st[1:2, :] * inv_count - mean * mean, 0.0)
        scale_v = g1_ref[...] * jax.lax.rsqrt(var + _EPS)
        shift_v = be1_ref[...] - mean * scale_v
        col_id = jax.lax.broadcasted_iota(jnp.int32, (th + 2, wp, cmid), 1)
        colmask = jnp.where((col_id >= _LP) & (col_id < w + _LP), 1.0, 0.0)
        scale = scale_v.reshape(1, 1, cmid) * colmask
        shift = shift_v.reshape(1, 1, cmid) * colmask
        zrow = jnp.zeros((1, wp, cmid), jnp.bfloat16)

        def tile(i, carry, top, bottom):
            s_acc, q_acc = carry
            slab = c1_ref[0, pl.ds(i * th, th + 2), :, :].astype(jnp.float32)
            hval = jnp.maximum(slab * scale + shift, 0.0).astype(jnp.bfloat16)
            if top:
                hval = jnp.concatenate([zrow, hval[1:]], axis=0)
            if bottom:
                hval = jnp.concatenate([hval[:th + 1], zrow], axis=0)
            taps = [hval[dy:dy + th, _LP - 1 + dx:_LP - 1 + dx + w, :]
                    for dy in range(3) for dx in range(3)]
            patches = jnp.concatenate(taps, axis=-1).reshape(th * w, 9 * cmid)
            res = jax.lax.dot_general(
                w_ref[...], patches,
                dimension_numbers=(((0,), (1,)), ((), ())),
                preferred_element_type=jnp.float32)
            s_acc = s_acc + jnp.sum(res, axis=1, keepdims=True)
            q_acc = q_acc + jnp.sum(res * res, axis=1, keepdims=True)
            out_ref[0, :, pl.ds(i * th * w, th * w)] = res.astype(jnp.bfloat16)
            return s_acc, q_acc

        zero = jnp.zeros((cout, 1), jnp.float32)
        carry = tile(0, (zero, zero), top=True, bottom=False)
        carry = jax.lax.fori_loop(
            1, n_h - 1,
            lambda i, c: tile(i, c, top=False, bottom=False),
            carry, unroll=2)
        s_acc, q_acc = tile(n_h - 1, carry, top=False, bottom=True)
        stats_ref[0, :, 0:1] = s_acc
        stats_ref[0, :, 1:2] = q_acc

    return kernel


def _make_bn2_relu_kernel(count):
    def kernel(c2_ref, st2_ref, g2_ref, be2_ref, out_ref):
        c = g2_ref.shape[0]
        inv_count = 1.0 / count
        st = jnp.sum(st2_ref[...], axis=0)
        mean = st[:, 0:1] * inv_count
        var = jnp.maximum(st[:, 1:2] * inv_count - mean * mean, 0.0)
        scale = g2_ref[...] * jax.lax.rsqrt(var + _EPS)
        shift = be2_ref[...] - mean * scale
        out_ref[0] = jnp.maximum(
            c2_ref[0].astype(jnp.float32) * scale.reshape(c, 1, 1)
            + shift.reshape(c, 1, 1), 0.0)
    return kernel


def kernel(x_nchw, w1, b1, g1, be1, w2, b2, g2, be2):
    n, cin, h, w = x_nchw.shape
    cout = w1.shape[-1]
    wp = _round_up(w + 2 * _LP, 16)
    th = 32 if (h % 32 == 0 and h >= 96) else (16 if (h % 16 == 0 and h >= 48) else 8)

    w1f = w1.reshape(9 * cin, cout).astype(jnp.bfloat16)
    w2f = w2.reshape(9 * cout, cout).astype(jnp.bfloat16)

    c1, st1 = pl.pallas_call(
        _make_conv1_kernel(h, w, cin, cout, wp, th),
        grid=(n,),
        out_shape=(jax.ShapeDtypeStruct((n, h + 2, wp, cout), jnp.bfloat16),
                   jax.ShapeDtypeStruct((n, 2, cout), jnp.float32)),
        in_specs=[
            pl.BlockSpec((1, cin, h, w), lambda b: (b, 0, 0, 0)),
            pl.BlockSpec((9 * cin, cout), lambda b: (0, 0)),
        ],
        out_specs=(
            pl.BlockSpec((1, h + 2, wp, cout), lambda b: (b, 0, 0, 0)),
            pl.BlockSpec((1, 2, cout), lambda b: (b, 0, 0)),
        ),
        scratch_shapes=[pltpu.VMEM((h + 2, wp, cin), jnp.bfloat16)],
        compiler_params=pltpu.CompilerParams(
            dimension_semantics=("parallel",),
            vmem_limit_bytes=_VMEM_LIMIT),
    )(x_nchw, w1f)

    c2, st2 = pl.pallas_call(
        _make_conv2_kernel(h, w, cout, cout, wp, th),
        grid=(n,),
        out_shape=(jax.ShapeDtypeStruct((n, cout, h * w), jnp.bfloat16),
                   jax.ShapeDtypeStruct((n, cout, 2), jnp.float32)),
        in_specs=[
            pl.BlockSpec((1, h + 2, wp, cout), lambda b: (b, 0, 0, 0)),
            pl.BlockSpec((9 * cout, cout), lambda b: (0, 0)),
            pl.BlockSpec((n, 2, cout), lambda b: (0, 0, 0)),
            pl.BlockSpec((1, cout), lambda b: (0, 0)),
            pl.BlockSpec((1, cout), lambda b: (0, 0)),
        ],
        out_specs=(
            pl.BlockSpec((1, cout, h * w), lambda b: (b, 0, 0)),
            pl.BlockSpec((1, cout, 2), lambda b: (b, 0, 0)),
        ),
        compiler_params=pltpu.CompilerParams(
            dimension_semantics=("parallel",),
            vmem_limit_bytes=_VMEM_LIMIT),
    )(c1, w2f, st1, g1.reshape(1, cout).astype(jnp.float32),
      be1.reshape(1, cout).astype(jnp.float32))

    c2_4d = c2.reshape(n, cout, h, w)

    n_s = 2 if h % 2 == 0 else 1
    h_blk = h // n_s
    return pl.pallas_call(
        _make_bn2_relu_kernel(float(n * h * w)),
        grid=(n, n_s),
        out_shape=jax.ShapeDtypeStruct((n, cout, h, w), jnp.float32),
        in_specs=[
            pl.BlockSpec((1, cout, h_blk, w), lambda b, j: (b, 0, j, 0)),
            pl.BlockSpec((n, cout, 2), lambda b, j: (0, 0, 0)),
            pl.BlockSpec((cout, 1), lambda b, j: (0, 0)),
            pl.BlockSpec((cout, 1), lambda b, j: (0, 0)),
        ],
        out_specs=pl.BlockSpec((1, cout, h_blk, w), lambda b, j: (b, 0, j, 0)),
        compiler_params=pltpu.CompilerParams(
            dimension_semantics=("parallel", "parallel"),
            vmem_limit_bytes=_VMEM_LIMIT),
    )(c2_4d, st2, g2.reshape(cout, 1).astype(jnp.float32),
      be2.reshape(cout, 1).astype(jnp.float32))
